# baseline (device time: 14733 ns/iter reference)
import jax
import jax.numpy as jnp
from jax import lax
from jax.experimental import pallas as pl
from jax.experimental.pallas import tpu as pltpu

N_DEV = 32


def kernel(x):
    m, n = x.shape

    def body(x_ref, out_ref, acc_ref, v_ref, comm_ref, send_sems, recv_sems):
        my = lax.axis_index("i")

        barrier = pltpu.get_barrier_semaphore()
        for j in range(1, N_DEV):
            src = lax.rem(my - j + N_DEV, N_DEV)
            pl.semaphore_signal(
                barrier, inc=1,
                device_id=(src,), device_id_type=pl.DeviceIdType.MESH,
            )

        xf = x_ref[...].astype(jnp.float32)
        acc_ref[...] = xf
        t = xf
        size = m
        while size > 1:
            half = size // 2
            t = t[:half] * t[half:size]
            size = half
        v_ref[...] = t

        s = 1
        while s < m:
            prev = acc_ref[pl.ds(0, m - s), :]
            cur = acc_ref[pl.ds(s, m - s), :]
            acc_ref[pl.ds(s, m - s), :] = cur * prev
            s *= 2

        pl.semaphore_wait(barrier, N_DEV - 1)

        rdmas = []
        for j in range(1, N_DEV):
            dst = lax.rem(my + j, N_DEV)
            rdma = pltpu.make_async_remote_copy(
                src_ref=v_ref,
                dst_ref=comm_ref.at[j],
                send_sem=send_sems.at[j],
                recv_sem=recv_sems.at[j],
                device_id=(dst,),
                device_id_type=pl.DeviceIdType.MESH,
            )
            rdma.start()
            rdmas.append(rdma)

        out_ref[...] = acc_ref[...].astype(jnp.bfloat16)

        for rdma in rdmas:
            rdma.wait()

        vals = comm_ref[:, 0, :]
        row = lax.broadcasted_iota(jnp.int32, (N_DEV, n), 0)
        srcidx = lax.rem(my - row + N_DEV, N_DEV)
        masked = jnp.where(srcidx < my, vals, jnp.ones_like(vals))
        size = N_DEV
        while size > 1:
            half = size // 2
            masked = masked[:half] * masked[half:size]
            size = half
        prefix = masked

        out_ref[...] = out_ref[...] * prefix.astype(jnp.bfloat16)

    return pl.pallas_call(
        body,
        out_shape=jax.ShapeDtypeStruct((m, n), jnp.bfloat16),
        in_specs=[pl.BlockSpec(memory_space=pltpu.VMEM)],
        out_specs=pl.BlockSpec(memory_space=pltpu.VMEM),
        scratch_shapes=[
            pltpu.VMEM((m, n), jnp.float32),
            pltpu.VMEM((1, n), jnp.float32),
            pltpu.VMEM((N_DEV, 1, n), jnp.float32),
            pltpu.SemaphoreType.DMA((N_DEV,)),
            pltpu.SemaphoreType.DMA((N_DEV,)),
        ],
        compiler_params=pltpu.CompilerParams(collective_id=0),
    )(x)
